# Initial kernel scaffold; baseline (speedup 1.0000x reference)
import jax
import jax.numpy as jnp
from jax.experimental import pallas as pl


def kernel(x, Wg, bg, We, be, Wc, bc):
    raise NotImplementedError("write your pallas kernel here")


# speedup vs baseline: 1.3092x; 1.3092x over previous
import jax
import jax.numpy as jnp
from jax.experimental import pallas as pl


def _moe_body(x_ref, WgT_ref, bg_ref, WeT_ref, be_ref, WcT_ref, bc_ref,
              logits_ref, sparse_ref, mixed_ref, full_ref):
    E = WeT_ref.shape[0]
    BT = x_ref.shape[0]
    xt = x_ref[...]

    gl = jnp.dot(xt, WgT_ref[...], preferred_element_type=jnp.float32)
    gl = gl + bg_ref[...][None, :]
    m = jnp.max(gl, axis=1, keepdims=True)
    eg = jnp.exp(gl - m)
    probs = eg / jnp.sum(eg, axis=1, keepdims=True)
    full_ref[...] = probs

    e_iota = jax.lax.broadcasted_iota(jnp.int32, (BT, E), 1)
    v1 = jnp.max(probs, axis=1, keepdims=True)
    i1 = jnp.min(jnp.where(probs == v1, e_iota, E), axis=1, keepdims=True)
    probs2 = jnp.where(e_iota == i1, -1.0, probs)
    v2 = jnp.max(probs2, axis=1, keepdims=True)
    i2 = jnp.min(jnp.where(probs2 == v2, e_iota, E), axis=1, keepdims=True)
    s = v1 + v2
    sparse = jnp.where(e_iota == i1, v1 / s, 0.0) + jnp.where(e_iota == i2, v2 / s, 0.0)
    sparse_ref[...] = sparse

    acc = jnp.zeros((BT, WeT_ref.shape[2]), jnp.float32)
    for e in range(E):
        h = jnp.dot(xt, WeT_ref[e], preferred_element_type=jnp.float32)
        h = jnp.maximum(h + be_ref[e][None, :], 0.0)
        acc = acc + sparse[:, e:e + 1] * h
    mixed_ref[...] = acc

    logits_ref[...] = (
        jnp.dot(acc, WcT_ref[...], preferred_element_type=jnp.float32)
        + bc_ref[...][None, :]
    )


def kernel(x, Wg, bg, We, be, Wc, bc):
    B, D = x.shape
    E, H, _ = We.shape
    C = Wc.shape[0]
    Cp = (C + 127) // 128 * 128

    WgT = Wg.T
    WeT = jnp.swapaxes(We, 1, 2)
    WcT = jnp.pad(Wc, ((0, Cp - C), (0, 0))).T
    bcp = jnp.pad(bc, (0, Cp - C))

    BT = 512 if B % 512 == 0 else B
    grid = (B // BT,)

    logits_p, sparse, mixed, full = pl.pallas_call(
        _moe_body,
        grid=grid,
        in_specs=[
            pl.BlockSpec((BT, D), lambda i: (i, 0)),
            pl.BlockSpec((D, E), lambda i: (0, 0)),
            pl.BlockSpec((E,), lambda i: (0,)),
            pl.BlockSpec((E, D, H), lambda i: (0, 0, 0)),
            pl.BlockSpec((E, H), lambda i: (0, 0)),
            pl.BlockSpec((H, Cp), lambda i: (0, 0)),
            pl.BlockSpec((Cp,), lambda i: (0,)),
        ],
        out_specs=[
            pl.BlockSpec((BT, Cp), lambda i: (i, 0)),
            pl.BlockSpec((BT, E), lambda i: (i, 0)),
            pl.BlockSpec((BT, H), lambda i: (i, 0)),
            pl.BlockSpec((BT, E), lambda i: (i, 0)),
        ],
        out_shape=[
            jax.ShapeDtypeStruct((B, Cp), jnp.float32),
            jax.ShapeDtypeStruct((B, E), jnp.float32),
            jax.ShapeDtypeStruct((B, H), jnp.float32),
            jax.ShapeDtypeStruct((B, E), jnp.float32),
        ],
    )(x, WgT, bg, WeT, be, WcT, bcp)

    return (logits_p[:, :C], sparse, mixed, full)


# speedup vs baseline: 1.3424x; 1.0253x over previous
import jax
import jax.numpy as jnp
from jax.experimental import pallas as pl


def _moe_body(x_ref, WgT_ref, bg_ref, WeT_ref, be_ref, WcT_ref, bc_ref,
              logits_ref, sparse_ref, mixed_ref, full_ref):
    E = WeT_ref.shape[0]
    BT = x_ref.shape[0]
    xt = x_ref[...]

    gl = jnp.dot(xt, WgT_ref[...], preferred_element_type=jnp.float32)
    gl = gl + bg_ref[...][None, :]
    m = jnp.max(gl, axis=1, keepdims=True)
    eg = jnp.exp(gl - m)
    probs = eg / jnp.sum(eg, axis=1, keepdims=True)
    full_ref[...] = probs

    e_iota = jax.lax.broadcasted_iota(jnp.int32, (BT, E), 1)
    v1 = jnp.max(probs, axis=1, keepdims=True)
    i1 = jnp.min(jnp.where(probs == v1, e_iota, E), axis=1, keepdims=True)
    probs2 = jnp.where(e_iota == i1, -1.0, probs)
    v2 = jnp.max(probs2, axis=1, keepdims=True)
    i2 = jnp.min(jnp.where(probs2 == v2, e_iota, E), axis=1, keepdims=True)
    s = v1 + v2
    sparse = jnp.where(e_iota == i1, v1 / s, 0.0) + jnp.where(e_iota == i2, v2 / s, 0.0)
    sparse_ref[...] = sparse

    xb = xt.astype(jnp.bfloat16)
    acc = jnp.zeros((BT, WeT_ref.shape[2]), jnp.float32)
    for e in range(E):
        h = jnp.dot(xb, WeT_ref[e], preferred_element_type=jnp.float32)
        h = jnp.maximum(h + be_ref[e][None, :], 0.0)
        acc = acc + sparse[:, e:e + 1] * h
    mixed_ref[...] = acc

    logits_ref[...] = (
        jnp.dot(acc.astype(jnp.bfloat16), WcT_ref[...],
                preferred_element_type=jnp.float32)
        + bc_ref[...][None, :]
    )


def kernel(x, Wg, bg, We, be, Wc, bc):
    B, D = x.shape
    E, H, _ = We.shape
    C = Wc.shape[0]
    Cp = (C + 127) // 128 * 128

    WgT = Wg.T
    WeT = jnp.swapaxes(We, 1, 2).astype(jnp.bfloat16)
    WcT = jnp.pad(Wc, ((0, Cp - C), (0, 0))).T.astype(jnp.bfloat16)
    bcp = jnp.pad(bc, (0, Cp - C))

    BT = 512 if B % 512 == 0 else B
    grid = (B // BT,)

    logits_p, sparse, mixed, full = pl.pallas_call(
        _moe_body,
        grid=grid,
        in_specs=[
            pl.BlockSpec((BT, D), lambda i: (i, 0)),
            pl.BlockSpec((D, E), lambda i: (0, 0)),
            pl.BlockSpec((E,), lambda i: (0,)),
            pl.BlockSpec((E, D, H), lambda i: (0, 0, 0)),
            pl.BlockSpec((E, H), lambda i: (0, 0)),
            pl.BlockSpec((H, Cp), lambda i: (0, 0)),
            pl.BlockSpec((Cp,), lambda i: (0,)),
        ],
        out_specs=[
            pl.BlockSpec((BT, Cp), lambda i: (i, 0)),
            pl.BlockSpec((BT, E), lambda i: (i, 0)),
            pl.BlockSpec((BT, H), lambda i: (i, 0)),
            pl.BlockSpec((BT, E), lambda i: (i, 0)),
        ],
        out_shape=[
            jax.ShapeDtypeStruct((B, Cp), jnp.float32),
            jax.ShapeDtypeStruct((B, E), jnp.float32),
            jax.ShapeDtypeStruct((B, H), jnp.float32),
            jax.ShapeDtypeStruct((B, E), jnp.float32),
        ],
    )(x, WgT, bg, WeT, be, WcT, bcp)

    return (logits_p[:, :C], sparse, mixed, full)


# speedup vs baseline: 1.3659x; 1.0175x over previous
import jax
import jax.numpy as jnp
from jax import lax
from jax.experimental import pallas as pl

_DN_T = (((1,), (1,)), ((), ()))


def _moe_body(x_ref, Wg_ref, bg_ref, We_ref, be_ref, Wc_ref, bc_ref,
              logits_ref, sparse_ref, mixed_ref, full_ref):
    E = We_ref.shape[0]
    BT = x_ref.shape[0]
    xt = x_ref[...]

    gl = lax.dot_general(xt, Wg_ref[...], _DN_T,
                         preferred_element_type=jnp.float32)
    gl = gl + bg_ref[...][None, :]
    m = jnp.max(gl, axis=1, keepdims=True)
    eg = jnp.exp(gl - m)
    probs = eg / jnp.sum(eg, axis=1, keepdims=True)
    full_ref[...] = probs

    e_iota = lax.broadcasted_iota(jnp.int32, (BT, E), 1)
    v1 = jnp.max(probs, axis=1, keepdims=True)
    i1 = jnp.min(jnp.where(probs == v1, e_iota, E), axis=1, keepdims=True)
    probs2 = jnp.where(e_iota == i1, -1.0, probs)
    v2 = jnp.max(probs2, axis=1, keepdims=True)
    i2 = jnp.min(jnp.where(probs2 == v2, e_iota, E), axis=1, keepdims=True)
    s = v1 + v2
    sparse = jnp.where(e_iota == i1, v1 / s, 0.0) + jnp.where(e_iota == i2, v2 / s, 0.0)
    sparse_ref[...] = sparse

    xb = xt.astype(jnp.bfloat16)
    acc = jnp.zeros((BT, We_ref.shape[1]), jnp.float32)
    for e in range(E):
        h = lax.dot_general(xb, We_ref[e], _DN_T,
                            preferred_element_type=jnp.float32)
        h = jnp.maximum(h + be_ref[e][None, :], 0.0)
        acc = acc + sparse[:, e:e + 1] * h
    mixed_ref[...] = acc

    logits_ref[...] = (
        lax.dot_general(acc.astype(jnp.bfloat16), Wc_ref[...], _DN_T,
                        preferred_element_type=jnp.float32)
        + bc_ref[...][None, :]
    )


def kernel(x, Wg, bg, We, be, Wc, bc):
    B, D = x.shape
    E, H, _ = We.shape
    C = Wc.shape[0]

    We_b = We.astype(jnp.bfloat16)
    Wc_b = Wc.astype(jnp.bfloat16)

    BT = 512 if B % 512 == 0 else B
    grid = (B // BT,)

    logits, sparse, mixed, full = pl.pallas_call(
        _moe_body,
        grid=grid,
        in_specs=[
            pl.BlockSpec((BT, D), lambda i: (i, 0)),
            pl.BlockSpec((E, D), lambda i: (0, 0)),
            pl.BlockSpec((E,), lambda i: (0,)),
            pl.BlockSpec((E, H, D), lambda i: (0, 0, 0)),
            pl.BlockSpec((E, H), lambda i: (0, 0)),
            pl.BlockSpec((C, H), lambda i: (0, 0)),
            pl.BlockSpec((C,), lambda i: (0,)),
        ],
        out_specs=[
            pl.BlockSpec((BT, C), lambda i: (i, 0)),
            pl.BlockSpec((BT, E), lambda i: (i, 0)),
            pl.BlockSpec((BT, H), lambda i: (i, 0)),
            pl.BlockSpec((BT, E), lambda i: (i, 0)),
        ],
        out_shape=[
            jax.ShapeDtypeStruct((B, C), jnp.float32),
            jax.ShapeDtypeStruct((B, E), jnp.float32),
            jax.ShapeDtypeStruct((B, H), jnp.float32),
            jax.ShapeDtypeStruct((B, E), jnp.float32),
        ],
    )(x, Wg, bg, We_b, be, Wc_b, bc)

    return (logits, sparse, mixed, full)


# speedup vs baseline: 1.3972x; 1.0229x over previous
import jax
import jax.numpy as jnp
from jax import lax
from jax.experimental import pallas as pl

_DN_T = (((1,), (1,)), ((), ()))


def _moe_body(x_ref, Wg_ref, bg_ref, We_ref, be_ref, Wc_ref, bc_ref,
              logits_ref, sparse_ref, mixed_ref, full_ref):
    E = We_ref.shape[0]
    BT = x_ref.shape[0]
    xt = x_ref[...]

    gl = lax.dot_general(xt, Wg_ref[...], _DN_T,
                         preferred_element_type=jnp.float32)
    gl = gl + bg_ref[...][None, :]
    m = jnp.max(gl, axis=1, keepdims=True)
    eg = jnp.exp(gl - m)
    probs = eg / jnp.sum(eg, axis=1, keepdims=True)
    full_ref[...] = probs

    e_iota = lax.broadcasted_iota(jnp.int32, (BT, E), 1)
    v1 = jnp.max(probs, axis=1, keepdims=True)
    i1 = jnp.min(jnp.where(probs == v1, e_iota, E), axis=1, keepdims=True)
    probs2 = jnp.where(e_iota == i1, -1.0, probs)
    v2 = jnp.max(probs2, axis=1, keepdims=True)
    i2 = jnp.min(jnp.where(probs2 == v2, e_iota, E), axis=1, keepdims=True)
    s = v1 + v2
    sparse = jnp.where(e_iota == i1, v1 / s, 0.0) + jnp.where(e_iota == i2, v2 / s, 0.0)
    sparse_ref[...] = sparse

    acc = jnp.zeros((BT, We_ref.shape[1]), jnp.float32)
    for e in range(E):
        h = lax.dot_general(xt, We_ref[e], _DN_T,
                            preferred_element_type=jnp.float32)
        h = jnp.maximum(h + be_ref[e][None, :], 0.0)
        acc = acc + sparse[:, e:e + 1] * h
    mixed_ref[...] = acc

    logits_ref[...] = (
        lax.dot_general(acc, Wc_ref[...], _DN_T,
                        preferred_element_type=jnp.float32)
        + bc_ref[...][None, :]
    )


def kernel(x, Wg, bg, We, be, Wc, bc):
    B, D = x.shape
    E, H, _ = We.shape
    C = Wc.shape[0]

    BT = 512 if B % 512 == 0 else B
    grid = (B // BT,)

    logits, sparse, mixed, full = pl.pallas_call(
        _moe_body,
        grid=grid,
        in_specs=[
            pl.BlockSpec((BT, D), lambda i: (i, 0)),
            pl.BlockSpec((E, D), lambda i: (0, 0)),
            pl.BlockSpec((E,), lambda i: (0,)),
            pl.BlockSpec((E, H, D), lambda i: (0, 0, 0)),
            pl.BlockSpec((E, H), lambda i: (0, 0)),
            pl.BlockSpec((C, H), lambda i: (0, 0)),
            pl.BlockSpec((C,), lambda i: (0,)),
        ],
        out_specs=[
            pl.BlockSpec((BT, C), lambda i: (i, 0)),
            pl.BlockSpec((BT, E), lambda i: (i, 0)),
            pl.BlockSpec((BT, H), lambda i: (i, 0)),
            pl.BlockSpec((BT, E), lambda i: (i, 0)),
        ],
        out_shape=[
            jax.ShapeDtypeStruct((B, C), jnp.float32),
            jax.ShapeDtypeStruct((B, E), jnp.float32),
            jax.ShapeDtypeStruct((B, H), jnp.float32),
            jax.ShapeDtypeStruct((B, E), jnp.float32),
        ],
    )(x, Wg, bg, We, be, Wc, bc)

    return (logits, sparse, mixed, full)
